# baseline (speedup 1.0000x reference)
import functools

import jax
import jax.numpy as jnp
from jax import lax
from jax.experimental import pallas as pl
from jax.experimental.pallas import tpu as pltpu
from jax.experimental.pallas import tpu_sc as plsc

N = 10000
E = 320000
D = 128
HD = D // 2
L = 3
BN_EPS = 1e-5

NC = 2
NS = 16
CH = 125
EPT = E // NS
NCH = EPT // CH
G = 16
NG = NCH // G
NBUF = 2

ROWS_SMALL = 624
ROWS_LAST = 640


def _stage_rows(s, fn):
    @pl.when(s < NS - 1)
    def _():
        fn(pl.ds(s * ROWS_SMALL, ROWS_SMALL))

    @pl.when(s == NS - 1)
    def _():
        fn(pl.ds((NS - 1) * ROWS_SMALL, ROWS_LAST))


def _sc_agg_body(hl_hbm, hr_hbm, src_hbm, dst_hbm, out_hbm,
                 src_v, dst_v, rows_v, h_sh, agg_sh, sem0, sem1, isem):
    c = lax.axis_index("c")
    s = lax.axis_index("s")

    @pl.when(c == 0)
    def _():
        def cp(rs):
            pltpu.sync_copy(hl_hbm.at[rs], h_sh.at[rs])
            pltpu.sync_copy(hl_hbm.at[rs], agg_sh.at[rs])
        _stage_rows(s, cp)

    @pl.when(c == 1)
    def _():
        def cp(rs):
            pltpu.sync_copy(hr_hbm.at[rs], h_sh.at[rs])
            pltpu.sync_copy(hr_hbm.at[rs], agg_sh.at[rs])
        _stage_rows(s, cp)

    pltpu.async_copy(src_hbm.at[pl.ds(s * NCH, G)], src_v.at[0], isem)
    pltpu.async_copy(dst_hbm.at[pl.ds(s * NCH, G)], dst_v.at[0], isem)
    plsc.subcore_barrier()

    def group_body(g, carry):
        gb = lax.rem(g, 2)
        pltpu.make_async_copy(src_hbm.at[pl.ds(0, G)], src_v.at[gb], isem).wait()
        pltpu.make_async_copy(dst_hbm.at[pl.ds(0, G)], dst_v.at[gb], isem).wait()

        @pl.when(g + 1 < NG)
        def _():
            nb = 1 - gb
            base = s * NCH + (g + 1) * G
            pltpu.async_copy(src_hbm.at[pl.ds(base, G)], src_v.at[nb], isem)
            pltpu.async_copy(dst_hbm.at[pl.ds(base, G)], dst_v.at[nb], isem)

        pltpu.async_copy(h_sh.at[src_v.at[gb, 0]], rows_v.at[0], sem0)
        pltpu.async_copy(h_sh.at[src_v.at[gb, 1]], rows_v.at[1], sem1)

        def pair_body(p, carry2):
            for b in range(NBUF):
                k = p * NBUF + b
                sem = sem0 if b == 0 else sem1
                buf = rows_v.at[b]
                pltpu.make_async_copy(h_sh.at[src_v.at[gb, k]], buf, sem).wait()
                pltpu.sync_copy(buf, agg_sh.at[dst_v.at[gb, k]], add=True)
                nxt = k + NBUF

                @pl.when(nxt < G)
                def _():
                    pltpu.async_copy(h_sh.at[src_v.at[gb, nxt]], buf, sem)
            return carry2

        lax.fori_loop(0, G // NBUF, pair_body, 0)
        return carry

    lax.fori_loop(0, NG, group_body, 0)

    plsc.subcore_barrier()

    def wb(rs):
        pltpu.sync_copy(agg_sh.at[rs],
                        out_hbm.at[rs, pl.ds(c * HD, HD)])
    _stage_rows(s, wb)


_sc_agg = functools.partial(
    pl.kernel,
    out_type=jax.ShapeDtypeStruct((N, D), jnp.float32),
    mesh=plsc.VectorSubcoreMesh(core_axis_name="c", subcore_axis_name="s"),
    scratch_types=[
        pltpu.VMEM((2, G, CH), jnp.int32),
        pltpu.VMEM((2, G, CH), jnp.int32),
        pltpu.VMEM((NBUF, CH, HD), jnp.float32),
        pltpu.VMEM_SHARED((N, HD), jnp.float32),
        pltpu.VMEM_SHARED((N, HD), jnp.float32),
        pltpu.SemaphoreType.DMA,
        pltpu.SemaphoreType.DMA,
        pltpu.SemaphoreType.DMA,
    ],
    compiler_params=pltpu.CompilerParams(use_tc_tiling_on_sc=False),
)(_sc_agg_body)


BLK = 1000
NBLK = N // BLK


def _mlp_body(p_ref, w1_ref, b1_ref, g_ref, be_ref,
              w2_ref, b2_ref, o_ref, ol_ref, or_ref, y_sc, stat_sc):
    p = pl.program_id(0)
    i = pl.program_id(1)
    base = pl.multiple_of(i * BLK, 8)

    @pl.when(p == 0)
    def _():
        out = p_ref[...]
        y = (jnp.dot(out, w1_ref[...], preferred_element_type=jnp.float32)
             + b1_ref[...])
        y_sc[pl.ds(base, BLK), :] = y

        @pl.when(i == 0)
        def _():
            stat_sc[...] = jnp.zeros_like(stat_sc)

        s = jnp.sum(y, axis=0, keepdims=True)
        ss = jnp.sum(y * y, axis=0, keepdims=True)
        stat_sc[...] += jnp.concatenate(
            [s, ss, jnp.zeros((6, D), jnp.float32)], axis=0)

    @pl.when(p == 1)
    def _():
        mu = stat_sc[0:1, :] / N
        var = stat_sc[1:2, :] / N - mu * mu
        inv = lax.rsqrt(var + BN_EPS) * g_ref[...]
        y = y_sc[pl.ds(base, BLK), :]
        z = jnp.maximum((y - mu) * inv + be_ref[...], 0.0)
        o = (jnp.dot(z, w2_ref[...], preferred_element_type=jnp.float32)
             + b2_ref[...])
        o = jnp.maximum(o, 0.0)
        o_ref[...] = o
        ol_ref[...] = o[:, :HD]
        or_ref[...] = o[:, HD:]


_in_row_spec = pl.BlockSpec((BLK, D), lambda p, i: ((1 - p) * i, 0))
_out_row_spec = pl.BlockSpec((BLK, D), lambda p, i: (p * i, 0))
_out_half_spec = pl.BlockSpec((BLK, HD), lambda p, i: (p * i, 0))
_full_spec = pl.BlockSpec((D, D), lambda p, i: (0, 0))
_vec_spec = pl.BlockSpec((1, D), lambda p, i: (0, 0))

_mlp = pl.pallas_call(
    _mlp_body,
    grid=(2, NBLK),
    in_specs=[_in_row_spec, _full_spec, _vec_spec,
              _vec_spec, _vec_spec, _full_spec, _vec_spec],
    out_specs=[_out_row_spec, _out_half_spec, _out_half_spec],
    out_shape=[jax.ShapeDtypeStruct((N, D), jnp.float32),
               jax.ShapeDtypeStruct((N, HD), jnp.float32),
               jax.ShapeDtypeStruct((N, HD), jnp.float32)],
    scratch_shapes=[pltpu.VMEM((N, D), jnp.float32),
                    pltpu.VMEM((8, D), jnp.float32)],
)


def kernel(x, edge_index, W1, b1, gamma, beta, W2, b2):
    src2d = edge_index[0].reshape(E // CH, CH)
    dst2d = edge_index[1].reshape(E // CH, CH)
    W1t = jnp.swapaxes(W1, 1, 2)
    W2t = jnp.swapaxes(W2, 1, 2)
    hl = x[:, :HD]
    hr = x[:, HD:]
    h = x
    for i in range(L):
        out = _sc_agg(hl, hr, src2d, dst2d)
        h, hl, hr = _mlp(out, W1t[i], b1[i].reshape(1, D),
                         gamma[i].reshape(1, D), beta[i].reshape(1, D),
                         W2t[i], b2[i].reshape(1, D))
    return h


# speedup vs baseline: 1.3767x; 1.3767x over previous
import functools

import jax
import jax.numpy as jnp
from jax import lax
from jax.experimental import pallas as pl
from jax.experimental.pallas import tpu as pltpu
from jax.experimental.pallas import tpu_sc as plsc

N = 10000
E = 320000
D = 128
L = 3
BN_EPS = 1e-5

NC = 2
NS = 16
NW = NC * NS
CH = 125
EPT = E // NW
NCH = EPT // CH
G = 16
NG = NCH // G
NBUF = 2

ROWS_SMALL = 624
ROWS_LAST = 640


def _sc_agg_body(h_hbm, src_hbm, dst_hbm, out_hbm,
                 src_v, dst_v, rows_v, zbuf, agg_sh, sem0, sem1, isem):
    c = lax.axis_index("c")
    s = lax.axis_index("s")
    w = c * NS + s

    pred_h = jnp.logical_or(jnp.logical_and(c == 0, s < NS // 2),
                            jnp.logical_and(c == 1, s >= NS // 2))

    @pl.when(pred_h)
    def _():
        @pl.when(s < NS - 1)
        def _():
            pltpu.sync_copy(h_hbm.at[pl.ds(s * ROWS_SMALL, ROWS_SMALL)],
                            agg_sh.at[pl.ds(s * ROWS_SMALL, ROWS_SMALL)])

        @pl.when(s == NS - 1)
        def _():
            pltpu.sync_copy(h_hbm.at[pl.ds((NS - 1) * ROWS_SMALL, ROWS_LAST)],
                            agg_sh.at[pl.ds((NS - 1) * ROWS_SMALL, ROWS_LAST)])

    @pl.when(jnp.logical_not(pred_h))
    def _():
        def zrow(t, carry):
            for m in range(D // 16):
                zbuf[t, pl.ds(16 * m, 16)] = jnp.zeros((16,), jnp.float32)
            return carry

        lax.fori_loop(0, 8, zrow, 0)

        def zfill(nblk, base):
            def fire(t, carry):
                pltpu.async_copy(zbuf, agg_sh.at[pl.ds(base + 8 * t, 8)], sem0)
                return carry

            def drain(t, carry):
                pltpu.make_async_copy(zbuf, agg_sh.at[pl.ds(0, 8)],
                                      sem0).wait()
                return carry

            lax.fori_loop(0, nblk, fire, 0)
            lax.fori_loop(0, nblk, drain, 0)

        @pl.when(s < NS - 1)
        def _():
            zfill(ROWS_SMALL // 8, s * ROWS_SMALL)

        @pl.when(s == NS - 1)
        def _():
            zfill(ROWS_LAST // 8, (NS - 1) * ROWS_SMALL)

    pltpu.async_copy(src_hbm.at[pl.ds(w * NCH, G)], src_v.at[0], isem)
    pltpu.async_copy(dst_hbm.at[pl.ds(w * NCH, G)], dst_v.at[0], isem)
    plsc.subcore_barrier()

    def group_body(g, carry):
        gb = lax.rem(g, 2)
        pltpu.make_async_copy(src_hbm.at[pl.ds(0, G)], src_v.at[gb], isem).wait()
        pltpu.make_async_copy(dst_hbm.at[pl.ds(0, G)], dst_v.at[gb], isem).wait()

        @pl.when(g + 1 < NG)
        def _():
            nb = 1 - gb
            base = w * NCH + (g + 1) * G
            pltpu.async_copy(src_hbm.at[pl.ds(base, G)], src_v.at[nb], isem)
            pltpu.async_copy(dst_hbm.at[pl.ds(base, G)], dst_v.at[nb], isem)

        pltpu.async_copy(h_hbm.at[src_v.at[gb, 0]], rows_v.at[0], sem0)
        pltpu.async_copy(h_hbm.at[src_v.at[gb, 1]], rows_v.at[1], sem1)

        def pair_body(p, carry2):
            for b in range(NBUF):
                k = p * NBUF + b
                sem = sem0 if b == 0 else sem1
                buf = rows_v.at[b]
                pltpu.make_async_copy(h_hbm.at[src_v.at[gb, k]], buf, sem).wait()
                pltpu.sync_copy(buf, agg_sh.at[dst_v.at[gb, k]], add=True)
                nxt = k + NBUF

                @pl.when(nxt < G)
                def _():
                    pltpu.async_copy(h_hbm.at[src_v.at[gb, nxt]], buf, sem)
            return carry2

        lax.fori_loop(0, G // NBUF, pair_body, 0)
        return carry

    lax.fori_loop(0, NG, group_body, 0)

    plsc.subcore_barrier()

    @pl.when(s < NS - 1)
    def _():
        pltpu.sync_copy(agg_sh.at[pl.ds(s * ROWS_SMALL, ROWS_SMALL)],
                        out_hbm.at[c, pl.ds(s * ROWS_SMALL, ROWS_SMALL)])

    @pl.when(s == NS - 1)
    def _():
        pltpu.sync_copy(agg_sh.at[pl.ds((NS - 1) * ROWS_SMALL, ROWS_LAST)],
                        out_hbm.at[c, pl.ds((NS - 1) * ROWS_SMALL, ROWS_LAST)])


_sc_agg = functools.partial(
    pl.kernel,
    out_type=jax.ShapeDtypeStruct((NC, N, D), jnp.float32),
    mesh=plsc.VectorSubcoreMesh(core_axis_name="c", subcore_axis_name="s"),
    scratch_types=[
        pltpu.VMEM((2, G, CH), jnp.int32),
        pltpu.VMEM((2, G, CH), jnp.int32),
        pltpu.VMEM((NBUF, CH, D), jnp.float32),
        pltpu.VMEM((8, D), jnp.float32),
        pltpu.VMEM_SHARED((N, D), jnp.float32),
        pltpu.SemaphoreType.DMA,
        pltpu.SemaphoreType.DMA,
        pltpu.SemaphoreType.DMA,
    ],
)(_sc_agg_body)


BLK = 1000
NBLK = N // BLK


def _mlp_body(p0_ref, p1_ref, w1_ref, b1_ref, g_ref, be_ref,
              w2_ref, b2_ref, o_ref, y_sc, stat_sc):
    p = pl.program_id(0)
    i = pl.program_id(1)
    base = pl.multiple_of(i * BLK, 8)

    @pl.when(p == 0)
    def _():
        out = p0_ref[...] + p1_ref[...]
        y = (jnp.dot(out, w1_ref[...], preferred_element_type=jnp.float32)
             + b1_ref[...])
        y_sc[pl.ds(base, BLK), :] = y

        @pl.when(i == 0)
        def _():
            stat_sc[...] = jnp.zeros_like(stat_sc)

        s = jnp.sum(y, axis=0, keepdims=True)
        ss = jnp.sum(y * y, axis=0, keepdims=True)
        stat_sc[...] += jnp.concatenate(
            [s, ss, jnp.zeros((6, D), jnp.float32)], axis=0)

    @pl.when(p == 1)
    def _():
        mu = stat_sc[0:1, :] / N
        var = stat_sc[1:2, :] / N - mu * mu
        inv = lax.rsqrt(var + BN_EPS) * g_ref[...]
        y = y_sc[pl.ds(base, BLK), :]
        z = jnp.maximum((y - mu) * inv + be_ref[...], 0.0)
        o = (jnp.dot(z, w2_ref[...], preferred_element_type=jnp.float32)
             + b2_ref[...])
        o_ref[...] = jnp.maximum(o, 0.0)


_in_row_spec = pl.BlockSpec((BLK, D), lambda p, i: ((1 - p) * i, 0))
_out_row_spec = pl.BlockSpec((BLK, D), lambda p, i: (p * i, 0))
_full_spec = pl.BlockSpec((D, D), lambda p, i: (0, 0))
_vec_spec = pl.BlockSpec((1, D), lambda p, i: (0, 0))

_mlp = pl.pallas_call(
    _mlp_body,
    grid=(2, NBLK),
    in_specs=[_in_row_spec, _in_row_spec, _full_spec, _vec_spec,
              _vec_spec, _vec_spec, _full_spec, _vec_spec],
    out_specs=_out_row_spec,
    out_shape=jax.ShapeDtypeStruct((N, D), jnp.float32),
    scratch_shapes=[pltpu.VMEM((N, D), jnp.float32),
                    pltpu.VMEM((8, D), jnp.float32)],
)


def kernel(x, edge_index, W1, b1, gamma, beta, W2, b2):
    src2d = edge_index[0].reshape(E // CH, CH)
    dst2d = edge_index[1].reshape(E // CH, CH)
    W1t = jnp.swapaxes(W1, 1, 2)
    W2t = jnp.swapaxes(W2, 1, 2)
    h = x
    for i in range(L):
        partials = _sc_agg(h, src2d, dst2d)
        h = _mlp(partials[0], partials[1], W1t[i], b1[i].reshape(1, D),
                 gamma[i].reshape(1, D), beta[i].reshape(1, D),
                 W2t[i], b2[i].reshape(1, D))
    return h


# speedup vs baseline: 1.4285x; 1.0376x over previous
import functools

import jax
import jax.numpy as jnp
from jax import lax
from jax.experimental import pallas as pl
from jax.experimental.pallas import tpu as pltpu
from jax.experimental.pallas import tpu_sc as plsc

N = 10000
E = 320000
D = 128
L = 3
BN_EPS = 1e-5

NC = 2
NS = 16
NW = NC * NS
CH = 125
EPT = E // NW
NCH = EPT // CH
G = 16
NG = NCH // G
NBUF = 2

ROWS_SMALL = 624
ROWS_LAST = 640


def _sc_agg_body(h_hbm, src_hbm, dst_hbm, out_hbm,
                 src_v, dst_v, rows_v, zbuf, agg_sh, sem0, sem1, isem):
    c = lax.axis_index("c")
    s = lax.axis_index("s")
    w = c * NS + s

    pred_h = jnp.logical_or(jnp.logical_and(c == 0, s < NS // 2),
                            jnp.logical_and(c == 1, s >= NS // 2))

    @pl.when(pred_h)
    def _():
        @pl.when(s < NS - 1)
        def _():
            pltpu.sync_copy(h_hbm.at[pl.ds(s * ROWS_SMALL, ROWS_SMALL)],
                            agg_sh.at[pl.ds(s * ROWS_SMALL, ROWS_SMALL)])

        @pl.when(s == NS - 1)
        def _():
            pltpu.sync_copy(h_hbm.at[pl.ds((NS - 1) * ROWS_SMALL, ROWS_LAST)],
                            agg_sh.at[pl.ds((NS - 1) * ROWS_SMALL, ROWS_LAST)])

    @pl.when(jnp.logical_not(pred_h))
    def _():
        def zrow(t, carry):
            for m in range(D // 16):
                zbuf[t, pl.ds(16 * m, 16)] = jnp.zeros((16,), jnp.float32)
            return carry

        lax.fori_loop(0, 8, zrow, 0)

        def zfill(nblk, base):
            def fire(t, carry):
                pltpu.async_copy(zbuf, agg_sh.at[pl.ds(base + 8 * t, 8)], sem0)
                return carry

            def drain(t, carry):
                pltpu.make_async_copy(zbuf, agg_sh.at[pl.ds(0, 8)],
                                      sem0).wait()
                return carry

            lax.fori_loop(0, nblk, fire, 0)
            lax.fori_loop(0, nblk, drain, 0)

        @pl.when(s < NS - 1)
        def _():
            zfill(ROWS_SMALL // 8, s * ROWS_SMALL)

        @pl.when(s == NS - 1)
        def _():
            zfill(ROWS_LAST // 8, (NS - 1) * ROWS_SMALL)

    pltpu.async_copy(src_hbm.at[pl.ds(w * NCH, G)], src_v.at[0], isem)
    pltpu.async_copy(dst_hbm.at[pl.ds(w * NCH, G)], dst_v.at[0], isem)
    plsc.subcore_barrier()

    def group_body(g, carry):
        gb = lax.rem(g, 2)
        pltpu.make_async_copy(src_hbm.at[pl.ds(0, G)], src_v.at[gb], isem).wait()
        pltpu.make_async_copy(dst_hbm.at[pl.ds(0, G)], dst_v.at[gb], isem).wait()

        @pl.when(g + 1 < NG)
        def _():
            nb = 1 - gb
            base = w * NCH + (g + 1) * G
            pltpu.async_copy(src_hbm.at[pl.ds(base, G)], src_v.at[nb], isem)
            pltpu.async_copy(dst_hbm.at[pl.ds(base, G)], dst_v.at[nb], isem)

        pltpu.async_copy(h_hbm.at[src_v.at[gb, 0]], rows_v.at[0], sem0)
        pltpu.async_copy(h_hbm.at[src_v.at[gb, 1]], rows_v.at[1], sem1)

        def pair_body(p, carry2):
            for b in range(NBUF):
                k = p * NBUF + b
                sem = sem0 if b == 0 else sem1
                buf = rows_v.at[b]
                pltpu.make_async_copy(h_hbm.at[src_v.at[gb, k]], buf, sem).wait()
                pltpu.sync_copy(buf, agg_sh.at[dst_v.at[gb, k]], add=True)
                nxt = k + NBUF

                @pl.when(nxt < G)
                def _():
                    pltpu.async_copy(h_hbm.at[src_v.at[gb, nxt]], buf, sem)
            return carry2

        lax.fori_loop(0, G // NBUF, pair_body, 0)
        return carry

    lax.fori_loop(0, NG, group_body, 0)

    plsc.subcore_barrier()

    @pl.when(s < NS - 1)
    def _():
        pltpu.sync_copy(agg_sh.at[pl.ds(s * ROWS_SMALL, ROWS_SMALL)],
                        out_hbm.at[c, pl.ds(s * ROWS_SMALL, ROWS_SMALL)])

    @pl.when(s == NS - 1)
    def _():
        pltpu.sync_copy(agg_sh.at[pl.ds((NS - 1) * ROWS_SMALL, ROWS_LAST)],
                        out_hbm.at[c, pl.ds((NS - 1) * ROWS_SMALL, ROWS_LAST)])


_sc_agg = functools.partial(
    pl.kernel,
    out_type=jax.ShapeDtypeStruct((NC, N, D), jnp.float32),
    mesh=plsc.VectorSubcoreMesh(core_axis_name="c", subcore_axis_name="s"),
    scratch_types=[
        pltpu.VMEM((2, G, CH), jnp.int32),
        pltpu.VMEM((2, G, CH), jnp.int32),
        pltpu.VMEM((NBUF, CH, D), jnp.float32),
        pltpu.VMEM((8, D), jnp.float32),
        pltpu.VMEM_SHARED((N, D), jnp.float32),
        pltpu.SemaphoreType.DMA,
        pltpu.SemaphoreType.DMA,
        pltpu.SemaphoreType.DMA,
    ],
)(_sc_agg_body)


BLK = 2000
NBLK = N // BLK


def _mlp_body(p0_ref, p1_ref, w1_ref, b1_ref, g_ref, be_ref,
              w2_ref, b2_ref, o_ref, y_sc, stat_sc):
    p = pl.program_id(0)
    i = pl.program_id(1)
    base = pl.multiple_of(i * BLK, 8)

    @pl.when(p == 0)
    def _():
        out = p0_ref[...] + p1_ref[...]
        y = (jnp.dot(out, w1_ref[...], preferred_element_type=jnp.float32)
             + b1_ref[...])
        y_sc[pl.ds(base, BLK), :] = y

        @pl.when(i == 0)
        def _():
            stat_sc[...] = jnp.zeros_like(stat_sc)

        stat_sc[0:1, :] += jnp.sum(y, axis=0, keepdims=True)
        stat_sc[1:2, :] += jnp.sum(y * y, axis=0, keepdims=True)

    @pl.when(p == 1)
    def _():
        mu = stat_sc[0:1, :] / N
        var = stat_sc[1:2, :] / N - mu * mu
        inv = lax.rsqrt(var + BN_EPS) * g_ref[...]
        y = y_sc[pl.ds(base, BLK), :]
        z = jnp.maximum((y - mu) * inv + be_ref[...], 0.0)
        o = (jnp.dot(z, w2_ref[...], preferred_element_type=jnp.float32)
             + b2_ref[...])
        o_ref[...] = jnp.maximum(o, 0.0)


_in_row_spec = pl.BlockSpec((BLK, D), lambda p, i: ((1 - p) * i, 0))
_out_row_spec = pl.BlockSpec((BLK, D), lambda p, i: (p * i, 0))
_full_spec = pl.BlockSpec((D, D), lambda p, i: (0, 0))
_vec_spec = pl.BlockSpec((1, D), lambda p, i: (0, 0))

_mlp = pl.pallas_call(
    _mlp_body,
    grid=(2, NBLK),
    in_specs=[_in_row_spec, _in_row_spec, _full_spec, _vec_spec,
              _vec_spec, _vec_spec, _full_spec, _vec_spec],
    out_specs=_out_row_spec,
    out_shape=jax.ShapeDtypeStruct((N, D), jnp.float32),
    scratch_shapes=[pltpu.VMEM((N, D), jnp.float32),
                    pltpu.VMEM((8, D), jnp.float32)],
)


def kernel(x, edge_index, W1, b1, gamma, beta, W2, b2):
    src2d = edge_index[0].reshape(E // CH, CH)
    dst2d = edge_index[1].reshape(E // CH, CH)
    W1t = jnp.swapaxes(W1, 1, 2)
    W2t = jnp.swapaxes(W2, 1, 2)
    h = x
    for i in range(L):
        partials = _sc_agg(h, src2d, dst2d)
        h = _mlp(partials[0], partials[1], W1t[i], b1[i].reshape(1, D),
                 gamma[i].reshape(1, D), beta[i].reshape(1, D),
                 W2t[i], b2[i].reshape(1, D))
    return h


# speedup vs baseline: 1.4372x; 1.0061x over previous
import functools

import jax
import jax.numpy as jnp
from jax import lax
from jax.experimental import pallas as pl
from jax.experimental.pallas import tpu as pltpu
from jax.experimental.pallas import tpu_sc as plsc

N = 10000
E = 320000
D = 128
L = 3
BN_EPS = 1e-5

NC = 2
NS = 16
NW = NC * NS
CH = 125
EPT = E // NW
NCH = EPT // CH
G = 16
NG = NCH // G
NBUF = 2

ROWS_SMALL = 624
ROWS_LAST = 640


def _sc_agg_body(h_hbm, src_hbm, dst_hbm, out_hbm,
                 src_v, dst_v, rows_v, zbuf, agg_sh, sem0, sem1, isem):
    c = lax.axis_index("c")
    s = lax.axis_index("s")
    w = c * NS + s

    pred_h = jnp.logical_or(jnp.logical_and(c == 0, s < NS // 2),
                            jnp.logical_and(c == 1, s >= NS // 2))

    @pl.when(pred_h)
    def _():
        @pl.when(s < NS - 1)
        def _():
            pltpu.sync_copy(h_hbm.at[pl.ds(s * ROWS_SMALL, ROWS_SMALL)],
                            agg_sh.at[pl.ds(s * ROWS_SMALL, ROWS_SMALL)])

        @pl.when(s == NS - 1)
        def _():
            pltpu.sync_copy(h_hbm.at[pl.ds((NS - 1) * ROWS_SMALL, ROWS_LAST)],
                            agg_sh.at[pl.ds((NS - 1) * ROWS_SMALL, ROWS_LAST)])

    @pl.when(jnp.logical_not(pred_h))
    def _():
        def zrow(t, carry):
            for m in range(D // 16):
                zbuf[t, pl.ds(16 * m, 16)] = jnp.zeros((16,), jnp.float32)
            return carry

        lax.fori_loop(0, 8, zrow, 0)

        def zfill(nblk, base):
            def fire(t, carry):
                pltpu.async_copy(zbuf, agg_sh.at[pl.ds(base + 8 * t, 8)], sem0)
                return carry

            def drain(t, carry):
                pltpu.make_async_copy(zbuf, agg_sh.at[pl.ds(0, 8)],
                                      sem0).wait()
                return carry

            lax.fori_loop(0, nblk, fire, 0)
            lax.fori_loop(0, nblk, drain, 0)

        @pl.when(s < NS - 1)
        def _():
            zfill(ROWS_SMALL // 8, s * ROWS_SMALL)

        @pl.when(s == NS - 1)
        def _():
            zfill(ROWS_LAST // 8, (NS - 1) * ROWS_SMALL)

    pltpu.async_copy(src_hbm.at[pl.ds(w * NCH, G)], src_v.at[0], isem)
    pltpu.async_copy(dst_hbm.at[pl.ds(w * NCH, G)], dst_v.at[0], isem)
    plsc.subcore_barrier()

    def group_body(g, carry):
        gb = lax.rem(g, 2)
        pltpu.make_async_copy(src_hbm.at[pl.ds(0, G)], src_v.at[gb], isem).wait()
        pltpu.make_async_copy(dst_hbm.at[pl.ds(0, G)], dst_v.at[gb], isem).wait()

        @pl.when(g + 1 < NG)
        def _():
            nb = 1 - gb
            base = w * NCH + (g + 1) * G
            pltpu.async_copy(src_hbm.at[pl.ds(base, G)], src_v.at[nb], isem)
            pltpu.async_copy(dst_hbm.at[pl.ds(base, G)], dst_v.at[nb], isem)

        pltpu.async_copy(h_hbm.at[src_v.at[gb, 0]], rows_v.at[0], sem0)
        pltpu.async_copy(h_hbm.at[src_v.at[gb, 1]], rows_v.at[1], sem1)

        def pair_body(p, carry2):
            for b in range(NBUF):
                k = p * NBUF + b
                sem = sem0 if b == 0 else sem1
                buf = rows_v.at[b]
                pltpu.make_async_copy(h_hbm.at[src_v.at[gb, k]], buf, sem).wait()
                pltpu.sync_copy(buf, agg_sh.at[dst_v.at[gb, k]], add=True)
                nxt = k + NBUF

                @pl.when(nxt < G)
                def _():
                    pltpu.async_copy(h_hbm.at[src_v.at[gb, nxt]], buf, sem)
            return carry2

        lax.fori_loop(0, G // NBUF, pair_body, 0)
        return carry

    lax.fori_loop(0, NG, group_body, 0)

    plsc.subcore_barrier()

    @pl.when(s < NS - 1)
    def _():
        pltpu.sync_copy(agg_sh.at[pl.ds(s * ROWS_SMALL, ROWS_SMALL)],
                        out_hbm.at[c, pl.ds(s * ROWS_SMALL, ROWS_SMALL)])

    @pl.when(s == NS - 1)
    def _():
        pltpu.sync_copy(agg_sh.at[pl.ds((NS - 1) * ROWS_SMALL, ROWS_LAST)],
                        out_hbm.at[c, pl.ds((NS - 1) * ROWS_SMALL, ROWS_LAST)])


_sc_agg = functools.partial(
    pl.kernel,
    out_type=jax.ShapeDtypeStruct((NC, N, D), jnp.float32),
    mesh=plsc.VectorSubcoreMesh(core_axis_name="c", subcore_axis_name="s"),
    scratch_types=[
        pltpu.VMEM((2, G, CH), jnp.int32),
        pltpu.VMEM((2, G, CH), jnp.int32),
        pltpu.VMEM((NBUF, CH, D), jnp.float32),
        pltpu.VMEM((8, D), jnp.float32),
        pltpu.VMEM_SHARED((N, D), jnp.float32),
        pltpu.SemaphoreType.DMA,
        pltpu.SemaphoreType.DMA,
        pltpu.SemaphoreType.DMA,
    ],
)(_sc_agg_body)


BLK = 5000
NBLK = N // BLK


def _mlp_body(p0_ref, p1_ref, w1_ref, b1_ref, g_ref, be_ref,
              w2_ref, b2_ref, o_ref, y_sc, stat_sc):
    p = pl.program_id(0)
    i = pl.program_id(1)
    base = pl.multiple_of(i * BLK, 8)

    @pl.when(p == 0)
    def _():
        out = p0_ref[...] + p1_ref[...]
        y = (jnp.dot(out, w1_ref[...], preferred_element_type=jnp.float32)
             + b1_ref[...])
        y_sc[pl.ds(base, BLK), :] = y

        @pl.when(i == 0)
        def _():
            stat_sc[...] = jnp.zeros_like(stat_sc)

        stat_sc[0:1, :] += jnp.sum(y, axis=0, keepdims=True)
        stat_sc[1:2, :] += jnp.sum(y * y, axis=0, keepdims=True)

    @pl.when(p == 1)
    def _():
        mu = stat_sc[0:1, :] / N
        var = stat_sc[1:2, :] / N - mu * mu
        inv = lax.rsqrt(var + BN_EPS) * g_ref[...]
        y = y_sc[pl.ds(base, BLK), :]
        z = jnp.maximum((y - mu) * inv + be_ref[...], 0.0)
        o = (jnp.dot(z, w2_ref[...], preferred_element_type=jnp.float32)
             + b2_ref[...])
        o_ref[...] = jnp.maximum(o, 0.0)


_in_row_spec = pl.BlockSpec((BLK, D), lambda p, i: ((1 - p) * i, 0))
_out_row_spec = pl.BlockSpec((BLK, D), lambda p, i: (p * i, 0))
_full_spec = pl.BlockSpec((D, D), lambda p, i: (0, 0))
_vec_spec = pl.BlockSpec((1, D), lambda p, i: (0, 0))

_mlp = pl.pallas_call(
    _mlp_body,
    grid=(2, NBLK),
    in_specs=[_in_row_spec, _in_row_spec, _full_spec, _vec_spec,
              _vec_spec, _vec_spec, _full_spec, _vec_spec],
    out_specs=_out_row_spec,
    out_shape=jax.ShapeDtypeStruct((N, D), jnp.float32),
    scratch_shapes=[pltpu.VMEM((N, D), jnp.float32),
                    pltpu.VMEM((8, D), jnp.float32)],
)


def kernel(x, edge_index, W1, b1, gamma, beta, W2, b2):
    src2d = edge_index[0].reshape(E // CH, CH)
    dst2d = edge_index[1].reshape(E // CH, CH)
    W1t = jnp.swapaxes(W1, 1, 2)
    W2t = jnp.swapaxes(W2, 1, 2)
    h = x
    for i in range(L):
        partials = _sc_agg(h, src2d, dst2d)
        h = _mlp(partials[0], partials[1], W1t[i], b1[i].reshape(1, D),
                 gamma[i].reshape(1, D), beta[i].reshape(1, D),
                 W2t[i], b2[i].reshape(1, D))
    return h


# speedup vs baseline: 1.4992x; 1.0431x over previous
import functools

import jax
import jax.numpy as jnp
from jax import lax
from jax.experimental import pallas as pl
from jax.experimental.pallas import tpu as pltpu
from jax.experimental.pallas import tpu_sc as plsc

N = 10000
E = 320000
D = 128
L = 3
BN_EPS = 1e-5

NC = 2
NS = 16
NW = NC * NS
CH = 125
EPT = E // NW
NCH = EPT // CH
G = 16
NG = NCH // G
NBUF = 2

ROWS_SMALL = 624
ROWS_LAST = 640


def _sc_agg_body(h_hbm, src_hbm, dst_hbm, out_hbm,
                 src_v, dst_v, rows_v, zbuf, agg_sh, sem0, sem1, isem):
    c = lax.axis_index("c")
    s = lax.axis_index("s")
    w = c * NS + s

    pred_h = jnp.logical_or(jnp.logical_and(c == 0, s < NS // 2),
                            jnp.logical_and(c == 1, s >= NS // 2))

    @pl.when(pred_h)
    def _():
        @pl.when(s < NS - 1)
        def _():
            pltpu.sync_copy(h_hbm.at[pl.ds(s * ROWS_SMALL, ROWS_SMALL)],
                            agg_sh.at[pl.ds(s * ROWS_SMALL, ROWS_SMALL)])

        @pl.when(s == NS - 1)
        def _():
            pltpu.sync_copy(h_hbm.at[pl.ds((NS - 1) * ROWS_SMALL, ROWS_LAST)],
                            agg_sh.at[pl.ds((NS - 1) * ROWS_SMALL, ROWS_LAST)])

    @pl.when(jnp.logical_not(pred_h))
    def _():
        def zrow(t, carry):
            for m in range(D // 16):
                zbuf[t, pl.ds(16 * m, 16)] = jnp.zeros((16,), jnp.float32)
            return carry

        lax.fori_loop(0, 8, zrow, 0)

        def zfill(nblk, base):
            def fire(t, carry):
                pltpu.async_copy(zbuf, agg_sh.at[pl.ds(base + 8 * t, 8)], sem0)
                return carry

            def drain(t, carry):
                pltpu.make_async_copy(zbuf, agg_sh.at[pl.ds(0, 8)],
                                      sem0).wait()
                return carry

            lax.fori_loop(0, nblk, fire, 0)
            lax.fori_loop(0, nblk, drain, 0)

        @pl.when(s < NS - 1)
        def _():
            zfill(ROWS_SMALL // 8, s * ROWS_SMALL)

        @pl.when(s == NS - 1)
        def _():
            zfill(ROWS_LAST // 8, (NS - 1) * ROWS_SMALL)

    pltpu.async_copy(src_hbm.at[pl.ds(w * NCH, G)], src_v.at[0], isem)
    pltpu.async_copy(dst_hbm.at[pl.ds(w * NCH, G)], dst_v.at[0], isem)
    pltpu.async_copy(src_hbm.at[pl.ds(w * NCH + G, G)], src_v.at[1], isem)
    pltpu.async_copy(dst_hbm.at[pl.ds(w * NCH + G, G)], dst_v.at[1], isem)
    plsc.subcore_barrier()
    pltpu.make_async_copy(src_hbm.at[pl.ds(0, G)], src_v.at[0], isem).wait()
    pltpu.make_async_copy(dst_hbm.at[pl.ds(0, G)], dst_v.at[0], isem).wait()
    pltpu.async_copy(h_hbm.at[src_v.at[0, 0]], rows_v.at[0], sem0)
    pltpu.async_copy(h_hbm.at[src_v.at[0, 1]], rows_v.at[1], sem1)

    def pair_body(p, carry):
        for b in range(NBUF):
            k = p * NBUF + b
            gb = lax.rem(k // G, 2)
            j = lax.rem(k, G)
            sem = sem0 if b == 0 else sem1
            buf = rows_v.at[b]
            pltpu.make_async_copy(h_hbm.at[src_v.at[gb, j]], buf, sem).wait()
            pltpu.sync_copy(buf, agg_sh.at[dst_v.at[gb, j]], add=True)
            nxt = k + NBUF

            @pl.when(nxt < NCH)
            def _():
                ng = nxt // G
                ngb = lax.rem(ng, 2)
                nj = lax.rem(nxt, G)

                @pl.when(nj == 0)
                def _():
                    pltpu.make_async_copy(src_hbm.at[pl.ds(0, G)],
                                          src_v.at[ngb], isem).wait()
                    pltpu.make_async_copy(dst_hbm.at[pl.ds(0, G)],
                                          dst_v.at[ngb], isem).wait()

                @pl.when(jnp.logical_and(nj == 1, ng + 1 < NG))
                def _():
                    base = w * NCH + (ng + 1) * G
                    pltpu.async_copy(src_hbm.at[pl.ds(base, G)],
                                     src_v.at[1 - ngb], isem)
                    pltpu.async_copy(dst_hbm.at[pl.ds(base, G)],
                                     dst_v.at[1 - ngb], isem)

                pltpu.async_copy(h_hbm.at[src_v.at[ngb, nj]], buf, sem)
        return carry

    lax.fori_loop(0, NCH // NBUF, pair_body, 0)

    plsc.subcore_barrier()

    @pl.when(s < NS - 1)
    def _():
        pltpu.sync_copy(agg_sh.at[pl.ds(s * ROWS_SMALL, ROWS_SMALL)],
                        out_hbm.at[c, pl.ds(s * ROWS_SMALL, ROWS_SMALL)])

    @pl.when(s == NS - 1)
    def _():
        pltpu.sync_copy(agg_sh.at[pl.ds((NS - 1) * ROWS_SMALL, ROWS_LAST)],
                        out_hbm.at[c, pl.ds((NS - 1) * ROWS_SMALL, ROWS_LAST)])


_sc_agg = functools.partial(
    pl.kernel,
    out_type=jax.ShapeDtypeStruct((NC, N, D), jnp.float32),
    mesh=plsc.VectorSubcoreMesh(core_axis_name="c", subcore_axis_name="s"),
    scratch_types=[
        pltpu.VMEM((2, G, CH), jnp.int32),
        pltpu.VMEM((2, G, CH), jnp.int32),
        pltpu.VMEM((NBUF, CH, D), jnp.float32),
        pltpu.VMEM((8, D), jnp.float32),
        pltpu.VMEM_SHARED((N, D), jnp.float32),
        pltpu.SemaphoreType.DMA,
        pltpu.SemaphoreType.DMA,
        pltpu.SemaphoreType.DMA,
    ],
)(_sc_agg_body)


BLK = 5000
NBLK = N // BLK


def _mlp_body(p0_ref, p1_ref, w1_ref, b1_ref, g_ref, be_ref,
              w2_ref, b2_ref, o_ref, y_sc, stat_sc):
    p = pl.program_id(0)
    i = pl.program_id(1)
    base = pl.multiple_of(i * BLK, 8)

    @pl.when(p == 0)
    def _():
        out = p0_ref[...] + p1_ref[...]
        y = (jnp.dot(out, w1_ref[...], preferred_element_type=jnp.float32)
             + b1_ref[...])
        y_sc[pl.ds(base, BLK), :] = y

        @pl.when(i == 0)
        def _():
            stat_sc[...] = jnp.zeros_like(stat_sc)

        stat_sc[0:1, :] += jnp.sum(y, axis=0, keepdims=True)
        stat_sc[1:2, :] += jnp.sum(y * y, axis=0, keepdims=True)

    @pl.when(p == 1)
    def _():
        mu = stat_sc[0:1, :] / N
        var = stat_sc[1:2, :] / N - mu * mu
        inv = lax.rsqrt(var + BN_EPS) * g_ref[...]
        y = y_sc[pl.ds(base, BLK), :]
        z = jnp.maximum((y - mu) * inv + be_ref[...], 0.0)
        o = (jnp.dot(z, w2_ref[...], preferred_element_type=jnp.float32)
             + b2_ref[...])
        o_ref[...] = jnp.maximum(o, 0.0)


_in_row_spec = pl.BlockSpec((BLK, D), lambda p, i: ((1 - p) * i, 0))
_out_row_spec = pl.BlockSpec((BLK, D), lambda p, i: (p * i, 0))
_full_spec = pl.BlockSpec((D, D), lambda p, i: (0, 0))
_vec_spec = pl.BlockSpec((1, D), lambda p, i: (0, 0))

_mlp = pl.pallas_call(
    _mlp_body,
    grid=(2, NBLK),
    in_specs=[_in_row_spec, _in_row_spec, _full_spec, _vec_spec,
              _vec_spec, _vec_spec, _full_spec, _vec_spec],
    out_specs=_out_row_spec,
    out_shape=jax.ShapeDtypeStruct((N, D), jnp.float32),
    scratch_shapes=[pltpu.VMEM((N, D), jnp.float32),
                    pltpu.VMEM((8, D), jnp.float32)],
)


def kernel(x, edge_index, W1, b1, gamma, beta, W2, b2):
    src2d = edge_index[0].reshape(E // CH, CH)
    dst2d = edge_index[1].reshape(E // CH, CH)
    W1t = jnp.swapaxes(W1, 1, 2)
    W2t = jnp.swapaxes(W2, 1, 2)
    h = x
    for i in range(L):
        partials = _sc_agg(h, src2d, dst2d)
        h = _mlp(partials[0], partials[1], W1t[i], b1[i].reshape(1, D),
                 gamma[i].reshape(1, D), beta[i].reshape(1, D),
                 W2t[i], b2[i].reshape(1, D))
    return h
